# Initial kernel scaffold; baseline (speedup 1.0000x reference)
import jax
import jax.numpy as jnp
from jax.experimental import pallas as pl


def kernel(seq, Wkv, W0, W1, W2, W3):
    raise NotImplementedError("write your pallas kernel here")


# speedup vs baseline: 14.8607x; 14.8607x over previous
import jax
import jax.numpy as jnp
from jax.experimental import pallas as pl
from jax.experimental.pallas import tpu as pltpu

_D = 128
_T = 32


def _body(seq_ref, wkv_ref, w0_ref, w1_ref, w2_ref, w3_ref,
          cum_ref, mem_ref, loss_ref, s_ref):
    d, t = _D, _T
    j = pl.program_id(1)
    nblk = pl.num_programs(1)

    @pl.when(j == 0)
    def _init():
        s_ref[...] = jnp.zeros_like(s_ref)

    x = seq_ref[0]
    wkv = wkv_ref[...]
    w0 = w0_ref[...]
    w1 = w1_ref[...]
    w2 = w2_ref[...]
    w3 = w3_ref[...]

    f32 = jnp.float32

    def mm(a, b):
        return jax.lax.dot_general(a, b, (((1,), (0,)), ((), ())),
                                   preferred_element_type=f32)

    def mm_tb(a, b):
        return jax.lax.dot_general(a, b, (((1,), (1,)), ((), ())),
                                   preferred_element_type=f32)

    def mm_ta(a, b):
        return jax.lax.dot_general(a, b, (((0,), (0,)), ((), ())),
                                   preferred_element_type=f32)

    kvt = mm_tb(wkv, x)
    kt = kvt[:d]
    vt = kvt[d:]

    def silu(h):
        s = jax.nn.sigmoid(h)
        return h * s, s

    h0 = mm(w0, kt)
    a0, s0 = silu(h0)
    h1 = mm(w1, a0)
    a1, s1 = silu(h1)
    h2 = mm(w2, a1)
    a2, s2 = silu(h2)
    pred = mm(w3, a2)

    e = pred - vt
    loss_ref[0, 0, 0, :] = jnp.mean(e * e, axis=0)

    dpred = (2.0 / d) * e

    def dsilu(h, s):
        return s * (1.0 + h * (1.0 - s))

    da2 = mm_ta(w3, dpred)
    dh2 = da2 * dsilu(h2, s2)
    da1 = mm_ta(w2, dh2)
    dh1 = da1 * dsilu(h1, s1)
    da0 = mm_ta(w1, dh1)
    dh0 = da0 * dsilu(h0, s0)

    iota_t = jax.lax.broadcasted_iota(jnp.int32, (t, d, t), 0)
    iota_s = jax.lax.broadcasted_iota(jnp.int32, (t, d, t), 2)
    mask3 = iota_s <= iota_t

    us = (dh0, dh1, dh2, dpred)
    vs = (kt, a0, a1, a2)
    ws = (w0, w1, w2, w3)

    for l in range(4):
        ut = us[l]
        vtl = vs[l]
        a3 = jnp.where(mask3, jnp.broadcast_to(ut[None], (t, d, t)), 0.0)
        amat = a3.reshape(t * d, t)
        p = mm_tb(amat, vtl).reshape(t, d, d)
        out = p + s_ref[l][None]
        cum_ref[l, 0] = out
        s_ref[l] = out[t - 1]

    @pl.when(j == nblk - 1)
    def _final():
        for l in range(4):
            mem_ref[l, 0] = ws[l] + s_ref[l]


def kernel(seq, Wkv, W0, W1, W2, W3):
    b, n, d = seq.shape
    t = _T
    nblk = n // t
    grid = (b, nblk)
    out_shape = (
        jax.ShapeDtypeStruct((4, b, n, d, d), jnp.float32),
        jax.ShapeDtypeStruct((4, b, d, d), jnp.float32),
        jax.ShapeDtypeStruct((b, nblk, 1, t), jnp.float32),
    )
    cum, nextmem, loss = pl.pallas_call(
        _body,
        grid=grid,
        in_specs=[
            pl.BlockSpec((1, t, d), lambda i, j: (i, j, 0)),
            pl.BlockSpec((2 * d, d), lambda i, j: (0, 0)),
            pl.BlockSpec((d, d), lambda i, j: (0, 0)),
            pl.BlockSpec((d, d), lambda i, j: (0, 0)),
            pl.BlockSpec((d, d), lambda i, j: (0, 0)),
            pl.BlockSpec((d, d), lambda i, j: (0, 0)),
        ],
        out_specs=[
            pl.BlockSpec((4, 1, t, d, d), lambda i, j: (0, i, j, 0, 0)),
            pl.BlockSpec((4, 1, d, d), lambda i, j: (0, i, 0, 0)),
            pl.BlockSpec((1, 1, 1, t), lambda i, j: (i, j, 0, 0)),
        ],
        out_shape=out_shape,
        scratch_shapes=[pltpu.VMEM((4, d, d), jnp.float32)],
        compiler_params=pltpu.CompilerParams(
            dimension_semantics=("arbitrary", "arbitrary"),
            vmem_limit_bytes=48 * 1024 * 1024,
        ),
    )(seq, Wkv, W0, W1, W2, W3)
    return cum, nextmem, loss.reshape(b * n)


# speedup vs baseline: 16.7622x; 1.1280x over previous
import jax
import jax.numpy as jnp
from jax.experimental import pallas as pl
from jax.experimental.pallas import tpu as pltpu

_D = 128
_T = 64


def _body(seq_ref, wkv_ref, w0_ref, w1_ref, w2_ref, w3_ref,
          cum_ref, mem_ref, loss_ref, s_ref):
    d, t = _D, _T
    j = pl.program_id(1)
    nblk = pl.num_programs(1)

    @pl.when(j == 0)
    def _init():
        s_ref[...] = jnp.zeros_like(s_ref)

    x = seq_ref[0]
    wkv = wkv_ref[...]
    w0 = w0_ref[...]
    w1 = w1_ref[...]
    w2 = w2_ref[...]
    w3 = w3_ref[...]

    f32 = jnp.float32

    def mm(a, b):
        return jax.lax.dot_general(a, b, (((1,), (0,)), ((), ())),
                                   preferred_element_type=f32)

    def mm_tb(a, b):
        return jax.lax.dot_general(a, b, (((1,), (1,)), ((), ())),
                                   preferred_element_type=f32)

    def mm_ta(a, b):
        return jax.lax.dot_general(a, b, (((0,), (0,)), ((), ())),
                                   preferred_element_type=f32)

    kvt = mm_tb(wkv, x)
    kt = kvt[:d]
    vt = kvt[d:]

    def silu(h):
        s = jax.nn.sigmoid(h)
        return h * s, s

    h0 = mm(w0, kt)
    a0, s0 = silu(h0)
    h1 = mm(w1, a0)
    a1, s1 = silu(h1)
    h2 = mm(w2, a1)
    a2, s2 = silu(h2)
    pred = mm(w3, a2)

    e = pred - vt
    loss_ref[0, 0, 0, :] = jnp.mean(e * e, axis=0)

    dpred = (2.0 / d) * e

    def dsilu(h, s):
        return s * (1.0 + h * (1.0 - s))

    da2 = mm_ta(w3, dpred)
    dh2 = da2 * dsilu(h2, s2)
    da1 = mm_ta(w2, dh2)
    dh1 = da1 * dsilu(h1, s1)
    da0 = mm_ta(w1, dh1)
    dh0 = da0 * dsilu(h0, s0)

    iota_t = jax.lax.broadcasted_iota(jnp.int32, (t, d, t), 0)
    iota_s = jax.lax.broadcasted_iota(jnp.int32, (t, d, t), 2)
    mask3 = iota_s <= iota_t

    us = (dh0, dh1, dh2, dpred)
    vs = (kt, a0, a1, a2)
    ws = (w0, w1, w2, w3)

    for l in range(4):
        ut = us[l]
        vtl = vs[l]
        a3 = jnp.where(mask3, jnp.broadcast_to(ut[None], (t, d, t)), 0.0)
        amat = a3.reshape(t * d, t)
        p = mm_tb(amat, vtl).reshape(t, d, d)
        out = p + s_ref[l][None]
        cum_ref[l, 0] = out
        s_ref[l] = out[t - 1]

    @pl.when(j == nblk - 1)
    def _final():
        for l in range(4):
            mem_ref[l, 0] = ws[l] + s_ref[l]


def kernel(seq, Wkv, W0, W1, W2, W3):
    b, n, d = seq.shape
    t = _T
    nblk = n // t
    grid = (b, nblk)
    out_shape = (
        jax.ShapeDtypeStruct((4, b, n, d, d), jnp.float32),
        jax.ShapeDtypeStruct((4, b, d, d), jnp.float32),
        jax.ShapeDtypeStruct((b, nblk, 1, t), jnp.float32),
    )
    cum, nextmem, loss = pl.pallas_call(
        _body,
        grid=grid,
        in_specs=[
            pl.BlockSpec((1, t, d), lambda i, j: (i, j, 0)),
            pl.BlockSpec((2 * d, d), lambda i, j: (0, 0)),
            pl.BlockSpec((d, d), lambda i, j: (0, 0)),
            pl.BlockSpec((d, d), lambda i, j: (0, 0)),
            pl.BlockSpec((d, d), lambda i, j: (0, 0)),
            pl.BlockSpec((d, d), lambda i, j: (0, 0)),
        ],
        out_specs=[
            pl.BlockSpec((4, 1, t, d, d), lambda i, j: (0, i, j, 0, 0)),
            pl.BlockSpec((4, 1, d, d), lambda i, j: (0, i, 0, 0)),
            pl.BlockSpec((1, 1, 1, t), lambda i, j: (i, j, 0, 0)),
        ],
        out_shape=out_shape,
        scratch_shapes=[pltpu.VMEM((4, d, d), jnp.float32)],
        compiler_params=pltpu.CompilerParams(
            dimension_semantics=("arbitrary", "arbitrary"),
            vmem_limit_bytes=48 * 1024 * 1024,
        ),
    )(seq, Wkv, W0, W1, W2, W3)
    return cum, nextmem, loss.reshape(b * n)


# speedup vs baseline: 16.8524x; 1.0054x over previous
import jax
import jax.numpy as jnp
from jax.experimental import pallas as pl
from jax.experimental.pallas import tpu as pltpu

_D = 128
_T = 64


def _body(seq_ref, wkv_ref, w0_ref, w1_ref, w2_ref, w3_ref,
          cum_ref, mem_ref, loss_ref, s_ref):
    d, t = _D, _T
    j = pl.program_id(1)
    nblk = pl.num_programs(1)

    @pl.when(j == 0)
    def _init():
        s_ref[...] = jnp.zeros_like(s_ref)

    x = seq_ref[0]
    wkv = wkv_ref[...]
    w0 = w0_ref[...]
    w1 = w1_ref[...]
    w2 = w2_ref[...]
    w3 = w3_ref[...]

    f32 = jnp.float32

    def mm(a, b):
        return jax.lax.dot_general(a, b, (((1,), (0,)), ((), ())),
                                   preferred_element_type=f32)

    def mm_tb(a, b):
        return jax.lax.dot_general(a, b, (((1,), (1,)), ((), ())),
                                   preferred_element_type=f32)

    def mm_ta(a, b):
        return jax.lax.dot_general(a, b, (((0,), (0,)), ((), ())),
                                   preferred_element_type=f32)

    kvt = mm_tb(wkv, x)
    kt = kvt[:d]
    vt = kvt[d:]

    def silu(h):
        s = jax.nn.sigmoid(h)
        return h * s, s

    h0 = mm(w0, kt)
    a0, s0 = silu(h0)
    h1 = mm(w1, a0)
    a1, s1 = silu(h1)
    h2 = mm(w2, a1)
    a2, s2 = silu(h2)
    pred = mm(w3, a2)

    e = pred - vt
    loss_ref[0, 0, 0, :] = jnp.mean(e * e, axis=0)

    dpred = (2.0 / d) * e

    def dsilu(h, s):
        return s * (1.0 + h * (1.0 - s))

    da2 = mm_ta(w3, dpred)
    dh2 = da2 * dsilu(h2, s2)
    da1 = mm_ta(w2, dh2)
    dh1 = da1 * dsilu(h1, s1)
    da0 = mm_ta(w1, dh1)
    dh0 = da0 * dsilu(h0, s0)

    iota_t = jax.lax.broadcasted_iota(jnp.int32, (t, d, t), 0)
    iota_s = jax.lax.broadcasted_iota(jnp.int32, (t, d, t), 2)
    mask3 = iota_s <= iota_t

    us = (dh0, dh1, dh2, dpred)
    vs = (kt, a0, a1, a2)
    ws = (w0, w1, w2, w3)

    for l in range(4):
        ut = us[l]
        vtl = vs[l]
        a3 = jnp.where(mask3, jnp.broadcast_to(ut[None], (t, d, t)), 0.0)
        amat = a3.reshape(t * d, t)
        p = mm_tb(amat, vtl).reshape(t, d, d)
        out = p + s_ref[l][None]
        cum_ref[l, 0] = out
        s_ref[l] = out[t - 1]

    @pl.when(j == nblk - 1)
    def _final():
        for l in range(4):
            mem_ref[l, 0] = ws[l] + s_ref[l]


def kernel(seq, Wkv, W0, W1, W2, W3):
    b, n, d = seq.shape
    t = _T
    nblk = n // t
    grid = (b, nblk)
    out_shape = (
        jax.ShapeDtypeStruct((4, b, n, d, d), jnp.float32),
        jax.ShapeDtypeStruct((4, b, d, d), jnp.float32),
        jax.ShapeDtypeStruct((b, nblk, 1, t), jnp.float32),
    )
    cum, nextmem, loss = pl.pallas_call(
        _body,
        grid=grid,
        in_specs=[
            pl.BlockSpec((1, t, d), lambda i, j: (i, j, 0)),
            pl.BlockSpec((2 * d, d), lambda i, j: (0, 0)),
            pl.BlockSpec((d, d), lambda i, j: (0, 0)),
            pl.BlockSpec((d, d), lambda i, j: (0, 0)),
            pl.BlockSpec((d, d), lambda i, j: (0, 0)),
            pl.BlockSpec((d, d), lambda i, j: (0, 0)),
        ],
        out_specs=[
            pl.BlockSpec((4, 1, t, d, d), lambda i, j: (0, i, j, 0, 0)),
            pl.BlockSpec((4, 1, d, d), lambda i, j: (0, i, 0, 0)),
            pl.BlockSpec((1, 1, 1, t), lambda i, j: (i, j, 0, 0)),
        ],
        out_shape=out_shape,
        scratch_shapes=[pltpu.VMEM((4, d, d), jnp.float32)],
        compiler_params=pltpu.CompilerParams(
            dimension_semantics=("parallel", "arbitrary"),
            vmem_limit_bytes=48 * 1024 * 1024,
        ),
    )(seq, Wkv, W0, W1, W2, W3)
    return cum, nextmem, loss.reshape(b * n)
